# baseline (speedup 1.0000x reference)
import functools

import jax
import jax.numpy as jnp
from jax import lax
from jax.experimental import pallas as pl
from jax.experimental.pallas import tpu as pltpu
from jax.experimental.pallas import tpu_sc as plsc

BATCH = 16384
EMBED = 32
NUM_CORES = 2
NUM_SUBCORES = 16
NUM_WORKERS = NUM_CORES * NUM_SUBCORES
ROWS_PER_WORKER = BATCH // NUM_WORKERS
CHUNK = 128
NUM_CHUNKS = ROWS_PER_WORKER // CHUNK
LANES = 16
GROUPS = ROWS_PER_WORKER // LANES


def _funk_body(uid_hbm, iid_hbm, p_hbm, q_hbm, out_hbm,
               uidx, iidx, pu, qi, yv, sem_p, sem_q):
  wid = lax.axis_index("s") * NUM_CORES + lax.axis_index("c")
  base = wid * ROWS_PER_WORKER

  pltpu.sync_copy(uid_hbm.at[pl.ds(base, ROWS_PER_WORKER)], uidx)
  pltpu.sync_copy(iid_hbm.at[pl.ds(base, ROWS_PER_WORKER)], iidx)

  copies = []
  for j in range(NUM_CHUNKS):
    sl = pl.ds(j * CHUNK, CHUNK)
    copies.append(pltpu.async_copy(p_hbm.at[uidx.at[sl]], pu.at[sl], sem_p))
    copies.append(pltpu.async_copy(q_hbm.at[iidx.at[sl]], qi.at[sl], sem_q))
  for c in copies:
    c.wait()

  lane = lax.iota(jnp.int32, LANES)
  masks = [lane == k for k in range(LANES)]

  def group(g, carry):
    r0 = g * LANES
    acc = jnp.zeros((LANES,), jnp.float32)
    for k in range(LANES):
      r = r0 + k
      m = (pu[r, pl.ds(0, LANES)] * qi[r, pl.ds(0, LANES)] +
           pu[r, pl.ds(LANES, LANES)] * qi[r, pl.ds(LANES, LANES)])
      s = jnp.sum(m, axis=0)
      acc = jnp.where(masks[k], jnp.broadcast_to(s, (LANES,)), acc)
    yv[pl.ds(r0, LANES)] = acc
    return carry

  lax.fori_loop(0, GROUPS, group, 0)

  pltpu.sync_copy(yv, out_hbm.at[pl.ds(base, ROWS_PER_WORKER)])


_funk = functools.partial(
    pl.kernel,
    out_type=jax.ShapeDtypeStruct((BATCH,), jnp.float32),
    mesh=plsc.VectorSubcoreMesh(core_axis_name="c", subcore_axis_name="s"),
    compiler_params=pltpu.CompilerParams(
        needs_layout_passes=False, use_tc_tiling_on_sc=False),
    scratch_types=[
        pltpu.VMEM((ROWS_PER_WORKER,), jnp.int32),
        pltpu.VMEM((ROWS_PER_WORKER,), jnp.int32),
        pltpu.VMEM((ROWS_PER_WORKER, EMBED), jnp.float32),
        pltpu.VMEM((ROWS_PER_WORKER, EMBED), jnp.float32),
        pltpu.VMEM((ROWS_PER_WORKER,), jnp.float32),
        pltpu.SemaphoreType.DMA,
        pltpu.SemaphoreType.DMA,
    ],
)(_funk_body)


@jax.jit
def kernel(user_ids, item_ids, P, Q):
  return _funk(user_ids.astype(jnp.int32), item_ids.astype(jnp.int32), P, Q)
